# baseline (speedup 1.0000x reference)
import jax
import jax.numpy as jnp
from jax import lax
from jax.experimental import pallas as pl
from jax.experimental.pallas import tpu as pltpu

B, TN, D = 16, 16, 1024
H, DH = 16, 64
LP = 2048
NC = 4
CP = LP // NC
NEG = -1e30
_DN = (((1,), (1,)), ((), ()))


def _proj_kernel(x_ref, wq_ref, wk_ref, wv_ref, bq_ref, bk_ref, bv_ref,
                 cos_ref, sin_ref, q_ref, k_ref, v_ref):
    x = x_ref[...]
    cos = cos_ref[...]
    sin = sin_ref[...]

    def rope(y):
        parts = []
        for h in range(H):
            yh = y[:, h * DH:(h + 1) * DH]
            rh = jnp.concatenate([yh[:, DH // 2:], yh[:, :DH // 2]], axis=-1)
            parts.append(yh * cos + rh * sin)
        return jnp.concatenate(parts, axis=-1)

    q = lax.dot_general(x, wq_ref[...], _DN, preferred_element_type=jnp.float32) + bq_ref[...]
    q_ref[...] = rope(q)
    k = lax.dot_general(x, wk_ref[...], _DN, preferred_element_type=jnp.float32) + bk_ref[...]
    k_ref[...] = rope(k)
    v_ref[...] = lax.dot_general(x, wv_ref[...], _DN, preferred_element_type=jnp.float32) + bv_ref[...]


def _attn_kernel(lens_ref, q_ref, kn_ref, vn_ref, kt_hbm, vt_hbm,
                 wo_ref, bo_ref, o_ref, ktb, vtb, k_sem, v_sem):
    b = pl.program_id(0)
    plen = lens_ref[0, b]
    nc = lens_ref[1, b]
    slot = lax.rem(b, 2)
    q = q_ref[0]
    kn = kn_ref[0]
    vn = vn_ref[0]

    def nch_of(bi):
        return lax.div(lens_ref[0, bi] + (CP - 1), CP)

    def issue(bi, sl):
        n = nch_of(bi)
        for c in range(NC):
            @pl.when(c < n)
            def _():
                cs = pl.ds(c * CP, CP)
                pltpu.make_async_copy(kt_hbm.at[bi, :, :, cs],
                                      ktb.at[sl, :, :, cs],
                                      k_sem.at[sl]).start()
                pltpu.make_async_copy(vt_hbm.at[bi, :, :, cs],
                                      vtb.at[sl, :, :, cs],
                                      v_sem.at[sl]).start()

    @pl.when(b == 0)
    def _prologue():
        vtb[...] = jnp.zeros_like(vtb)
        issue(0, 0)

    n_cur = nch_of(b)
    for c in range(NC):
        @pl.when(c < n_cur)
        def _():
            cs = pl.ds(c * CP, CP)
            pltpu.make_async_copy(kt_hbm.at[b, :, :, cs],
                                  ktb.at[slot, :, :, cs],
                                  k_sem.at[slot]).wait()
            pltpu.make_async_copy(vt_hbm.at[b, :, :, cs],
                                  vtb.at[slot, :, :, cs],
                                  v_sem.at[slot]).wait()

    @pl.when(b + 1 < B)
    def _prefetch_next():
        issue(b + 1, 1 - slot)

    ncol = lax.broadcasted_iota(jnp.int32, (1, TN), 1)
    nbias = jnp.where(ncol < nc, 0.0, NEG)
    scale = 1.0 / 8.0
    rowid = lax.broadcasted_iota(jnp.int32, (TN, 1), 0)

    def body(W):
        if W > 0:
            pvalid = lax.broadcasted_iota(jnp.int32, (1, W), 1) < plen

        def scores_of(h):
            sl = slice(h * DH, (h + 1) * DH)
            qh = q[:, sl] * scale
            sn = lax.dot_general(qh, kn[:, sl], _DN,
                                 preferred_element_type=jnp.float32) + nbias
            if W == 0:
                return sn
            sp = jnp.dot(qh, ktb[slot, h, :, :W],
                         preferred_element_type=jnp.float32)
            sp = jnp.where(pvalid, sp, NEG)
            return jnp.concatenate([sp, sn], axis=1)

        def out_of(h, p, denom):
            sl = slice(h * DH, (h + 1) * DH)
            on = jnp.dot(p[:, W:].astype(jnp.bfloat16),
                         vn[:, sl].astype(jnp.bfloat16),
                         preferred_element_type=jnp.float32)
            if W > 0:
                on = on + lax.dot_general(
                    p[:, :W].astype(jnp.bfloat16),
                    vtb[slot, h, :, :W].astype(jnp.bfloat16), _DN,
                    preferred_element_type=jnp.float32)
            return on * (1.0 / denom)

        outs = []
        for h0 in range(0, H, 2):
            s_a = scores_of(h0)
            s_b = scores_of(h0 + 1)
            m_a = jnp.max(s_a, axis=-1, keepdims=True)
            m_b = jnp.max(s_b, axis=-1, keepdims=True)
            p_a = jnp.exp(s_a - m_a)
            p_b = jnp.exp(s_b - m_b)
            d_a = jnp.sum(p_a, axis=-1, keepdims=True)
            d_b = jnp.sum(p_b, axis=-1, keepdims=True)
            outs.append(out_of(h0, p_a, d_a))
            outs.append(out_of(h0 + 1, p_b, d_b))

        of = jnp.concatenate(outs, axis=-1)
        out = lax.dot_general(of, wo_ref[...], _DN,
                              preferred_element_type=jnp.float32) + bo_ref[...]
        o_ref[0] = jnp.where(rowid < nc, out, 0.0)

    for w in range(NC + 1):
        @pl.when(n_cur == w)
        def _(w=w):
            body(w * CP)


def kernel(x_new, rotary_cos, rotary_sin, past_k, past_v,
           Wq, bq, Wk, bk, Wv, bv, Wo, bo,
           past_lengths, new_token_counts, valid_new_mask):
    xf = x_new.reshape(B * TN, D)
    cosf = jnp.tile(rotary_cos.reshape(TN, DH), (B, 1))
    sin1 = rotary_sin.reshape(TN, DH)
    sinf = jnp.tile(jnp.concatenate([-sin1[:, :DH // 2], sin1[:, DH // 2:]],
                                    axis=-1), (B, 1))

    q, kn, vn = pl.pallas_call(
        _proj_kernel,
        out_shape=[jax.ShapeDtypeStruct((B * TN, D), jnp.float32)] * 3,
        name="qkv_rope_proj",
    )(xf, Wq, Wk, Wv, bq.reshape(1, D), bk.reshape(1, D), bv.reshape(1, D),
      cosf, sinf)

    lens = jnp.stack([past_lengths, new_token_counts]).astype(jnp.int32)

    out = pl.pallas_call(
        _attn_kernel,
        grid_spec=pltpu.PrefetchScalarGridSpec(
            num_scalar_prefetch=1,
            grid=(B,),
            in_specs=[
                pl.BlockSpec((1, TN, D), lambda b, lens: (b, 0, 0)),
                pl.BlockSpec((1, TN, D), lambda b, lens: (b, 0, 0)),
                pl.BlockSpec((1, TN, D), lambda b, lens: (b, 0, 0)),
                pl.BlockSpec(memory_space=pl.ANY),
                pl.BlockSpec(memory_space=pl.ANY),
                pl.BlockSpec((D, D), lambda b, lens: (0, 0)),
                pl.BlockSpec((1, D), lambda b, lens: (0, 0)),
            ],
            out_specs=pl.BlockSpec((1, TN, D), lambda b, lens: (b, 0, 0)),
            scratch_shapes=[
                pltpu.VMEM((2, H, DH, LP), jnp.float32),
                pltpu.VMEM((2, H, DH, LP), jnp.float32),
                pltpu.SemaphoreType.DMA((2,)),
                pltpu.SemaphoreType.DMA((2,)),
            ],
        ),
        out_shape=jax.ShapeDtypeStruct((B, TN, D), jnp.float32),
        compiler_params=pltpu.CompilerParams(
            dimension_semantics=("arbitrary",),
            vmem_limit_bytes=50 * 1024 * 1024,
        ),
        name="ragged_attn",
    )(lens, q.reshape(B, TN, D), kn.reshape(B, TN, D), vn.reshape(B, TN, D),
      jnp.swapaxes(past_k, 2, 3), jnp.swapaxes(past_v, 2, 3),
      Wo, bo.reshape(1, D))

    return out


# speedup vs baseline: 1.0056x; 1.0056x over previous
import jax
import jax.numpy as jnp
from jax import lax
from jax.experimental import pallas as pl
from jax.experimental.pallas import tpu as pltpu

B, TN, D = 16, 16, 1024
H, DH = 16, 64
LP = 2048
NC = 4
CP = LP // NC
NEG = -1e30
_DN = (((1,), (1,)), ((), ()))


def _proj_kernel(x_ref, wq_ref, wk_ref, wv_ref, bq_ref, bk_ref, bv_ref,
                 cos_ref, sin_ref, q_ref, k_ref, v_ref):
    x = x_ref[...]
    cos = cos_ref[...]
    sin = sin_ref[...]

    def rope(y):
        parts = []
        for h in range(H):
            yh = y[:, h * DH:(h + 1) * DH]
            rh = jnp.concatenate([yh[:, DH // 2:], yh[:, :DH // 2]], axis=-1)
            parts.append(yh * cos + rh * sin)
        return jnp.concatenate(parts, axis=-1)

    q = lax.dot_general(x, wq_ref[...], _DN, preferred_element_type=jnp.float32) + bq_ref[...]
    q_ref[...] = rope(q)
    k = lax.dot_general(x, wk_ref[...], _DN, preferred_element_type=jnp.float32) + bk_ref[...]
    k_ref[...] = rope(k)
    v_ref[...] = lax.dot_general(x, wv_ref[...], _DN, preferred_element_type=jnp.float32) + bv_ref[...]


def _attn_kernel(lens_ref, q_ref, kn_ref, vn_ref, kt_hbm, vt_hbm,
                 wo_ref, bo_ref, o_ref, ktb, vtb, k_sem, v_sem):
    b = pl.program_id(0)
    plen = lens_ref[0, b]
    nc = lens_ref[1, b]
    slot = lax.rem(b, 2)
    q = q_ref[0]
    kn = kn_ref[0]
    vn = vn_ref[0]

    def nch_of(bi):
        return lax.div(lens_ref[0, bi] + (CP - 1), CP)

    def issue(bi, sl):
        n = nch_of(bi)
        for c in range(NC):
            @pl.when(c < n)
            def _():
                cs = pl.ds(c * CP, CP)
                pltpu.make_async_copy(kt_hbm.at[bi, :, :, cs],
                                      ktb.at[sl, :, :, cs],
                                      k_sem.at[sl]).start()
                pltpu.make_async_copy(vt_hbm.at[bi, :, :, cs],
                                      vtb.at[sl, :, :, cs],
                                      v_sem.at[sl]).start()

    @pl.when(b == 0)
    def _prologue():
        vtb[...] = jnp.zeros_like(vtb)
        issue(0, 0)

    n_cur = nch_of(b)
    for c in range(NC):
        @pl.when(c < n_cur)
        def _():
            cs = pl.ds(c * CP, CP)
            pltpu.make_async_copy(kt_hbm.at[b, :, :, cs],
                                  ktb.at[slot, :, :, cs],
                                  k_sem.at[slot]).wait()
            pltpu.make_async_copy(vt_hbm.at[b, :, :, cs],
                                  vtb.at[slot, :, :, cs],
                                  v_sem.at[slot]).wait()

    @pl.when(b + 1 < B)
    def _prefetch_next():
        issue(b + 1, 1 - slot)

    ncol = lax.broadcasted_iota(jnp.int32, (1, TN), 1)
    nbias = jnp.where(ncol < nc, 0.0, NEG)
    scale = 1.0 / 8.0
    rowid = lax.broadcasted_iota(jnp.int32, (TN, 1), 0)

    def body(W):
        if W > 0:
            pvalid = lax.broadcasted_iota(jnp.int32, (1, W), 1) < plen

        def scores_of(h):
            sl = slice(h * DH, (h + 1) * DH)
            qh = q[:, sl] * scale
            sn = lax.dot_general(qh, kn[:, sl], _DN,
                                 preferred_element_type=jnp.float32) + nbias
            if W == 0:
                return sn
            sp = jnp.dot(qh, ktb[slot, h, :, :W],
                         preferred_element_type=jnp.float32)
            sp = jnp.where(pvalid, sp, NEG)
            return jnp.concatenate([sp, sn], axis=1)

        def out_of(h, p, denom):
            sl = slice(h * DH, (h + 1) * DH)
            on = jnp.dot(p[:, W:], vn[:, sl],
                         preferred_element_type=jnp.float32)
            if W > 0:
                on = on + lax.dot_general(p[:, :W], vtb[slot, h, :, :W], _DN,
                                          preferred_element_type=jnp.float32)
            return on * (1.0 / denom)

        outs = []
        for h0 in range(0, H, 2):
            s_a = scores_of(h0)
            s_b = scores_of(h0 + 1)
            m_a = jnp.max(s_a, axis=-1, keepdims=True)
            m_b = jnp.max(s_b, axis=-1, keepdims=True)
            p_a = jnp.exp(s_a - m_a)
            p_b = jnp.exp(s_b - m_b)
            d_a = jnp.sum(p_a, axis=-1, keepdims=True)
            d_b = jnp.sum(p_b, axis=-1, keepdims=True)
            outs.append(out_of(h0, p_a, d_a))
            outs.append(out_of(h0 + 1, p_b, d_b))

        of = jnp.concatenate(outs, axis=-1)
        out = lax.dot_general(of, wo_ref[...], _DN,
                              preferred_element_type=jnp.float32) + bo_ref[...]
        o_ref[0] = jnp.where(rowid < nc, out, 0.0)

    for w in range(NC + 1):
        @pl.when(n_cur == w)
        def _(w=w):
            body(w * CP)


def kernel(x_new, rotary_cos, rotary_sin, past_k, past_v,
           Wq, bq, Wk, bk, Wv, bv, Wo, bo,
           past_lengths, new_token_counts, valid_new_mask):
    xf = x_new.reshape(B * TN, D)
    cosf = jnp.tile(rotary_cos.reshape(TN, DH), (B, 1))
    sin1 = rotary_sin.reshape(TN, DH)
    sinf = jnp.tile(jnp.concatenate([-sin1[:, :DH // 2], sin1[:, DH // 2:]],
                                    axis=-1), (B, 1))

    q, kn, vn = pl.pallas_call(
        _proj_kernel,
        out_shape=[jax.ShapeDtypeStruct((B * TN, D), jnp.float32)] * 3,
        name="qkv_rope_proj",
    )(xf, Wq, Wk, Wv, bq.reshape(1, D), bk.reshape(1, D), bv.reshape(1, D),
      cosf, sinf)

    lens = jnp.stack([past_lengths, new_token_counts]).astype(jnp.int32)

    out = pl.pallas_call(
        _attn_kernel,
        grid_spec=pltpu.PrefetchScalarGridSpec(
            num_scalar_prefetch=1,
            grid=(B,),
            in_specs=[
                pl.BlockSpec((1, TN, D), lambda b, lens: (b, 0, 0)),
                pl.BlockSpec((1, TN, D), lambda b, lens: (b, 0, 0)),
                pl.BlockSpec((1, TN, D), lambda b, lens: (b, 0, 0)),
                pl.BlockSpec(memory_space=pl.ANY),
                pl.BlockSpec(memory_space=pl.ANY),
                pl.BlockSpec((D, D), lambda b, lens: (0, 0)),
                pl.BlockSpec((1, D), lambda b, lens: (0, 0)),
            ],
            out_specs=pl.BlockSpec((1, TN, D), lambda b, lens: (b, 0, 0)),
            scratch_shapes=[
                pltpu.VMEM((2, H, DH, LP), jnp.float32),
                pltpu.VMEM((2, H, DH, LP), jnp.float32),
                pltpu.SemaphoreType.DMA((2,)),
                pltpu.SemaphoreType.DMA((2,)),
            ],
        ),
        out_shape=jax.ShapeDtypeStruct((B, TN, D), jnp.float32),
        compiler_params=pltpu.CompilerParams(
            dimension_semantics=("arbitrary",),
            vmem_limit_bytes=50 * 1024 * 1024,
        ),
        name="ragged_attn",
    )(lens, q.reshape(B, TN, D), kn.reshape(B, TN, D), vn.reshape(B, TN, D),
      jnp.swapaxes(past_k, 2, 3), jnp.swapaxes(past_v, 2, 3),
      Wo, bo.reshape(1, D))

    return out


# speedup vs baseline: 1.0160x; 1.0104x over previous
import jax
import jax.numpy as jnp
from jax import lax
from jax.experimental import pallas as pl
from jax.experimental.pallas import tpu as pltpu

B, TN, D = 16, 16, 1024
H, DH = 16, 64
LP = 2048
NC = 4
CP = LP // NC
NEG = -1e30
_DN = (((1,), (1,)), ((), ()))


def _proj_kernel(x_ref, wq_ref, wk_ref, wv_ref, bq_ref, bk_ref, bv_ref,
                 cos_ref, sin_ref, q_ref, k_ref, v_ref):
    x = x_ref[...]
    cos = cos_ref[...]
    sin = sin_ref[...]

    def rope(y):
        parts = []
        for h in range(y.shape[1] // DH):
            yh = y[:, h * DH:(h + 1) * DH]
            rh = jnp.concatenate([yh[:, DH // 2:], yh[:, :DH // 2]], axis=-1)
            parts.append(yh * cos + rh * sin)
        return jnp.concatenate(parts, axis=-1)

    q = lax.dot_general(x, wq_ref[...], _DN, preferred_element_type=jnp.float32) + bq_ref[...]
    q_ref[...] = rope(q)
    k = lax.dot_general(x, wk_ref[...], _DN, preferred_element_type=jnp.float32) + bk_ref[...]
    k_ref[...] = rope(k)
    v_ref[...] = lax.dot_general(x, wv_ref[...], _DN, preferred_element_type=jnp.float32) + bv_ref[...]


def _attn_kernel(lens_ref, q_ref, kn_ref, vn_ref, kt_hbm, vt_hbm,
                 wo_ref, bo_ref, o_ref, ktb, vtb, k_sem, v_sem):
    b = pl.program_id(0)
    plen = lens_ref[0, b]
    nc = lens_ref[1, b]
    slot = lax.rem(b, 2)
    q = q_ref[0]
    kn = kn_ref[0]
    vn = vn_ref[0]

    def nch_of(bi):
        return lax.div(lens_ref[0, bi] + (CP - 1), CP)

    def issue(bi, sl):
        n = nch_of(bi)
        for c in range(NC):
            @pl.when(c < n)
            def _():
                cs = pl.ds(c * CP, CP)
                pltpu.make_async_copy(kt_hbm.at[bi, :, :, cs],
                                      ktb.at[sl, :, :, cs],
                                      k_sem.at[sl]).start()
                pltpu.make_async_copy(vt_hbm.at[bi, :, :, cs],
                                      vtb.at[sl, :, :, cs],
                                      v_sem.at[sl]).start()

    @pl.when(b == 0)
    def _prologue():
        vtb[...] = jnp.zeros_like(vtb)
        issue(0, 0)

    n_cur = nch_of(b)
    for c in range(NC):
        @pl.when(c < n_cur)
        def _():
            cs = pl.ds(c * CP, CP)
            pltpu.make_async_copy(kt_hbm.at[b, :, :, cs],
                                  ktb.at[slot, :, :, cs],
                                  k_sem.at[slot]).wait()
            pltpu.make_async_copy(vt_hbm.at[b, :, :, cs],
                                  vtb.at[slot, :, :, cs],
                                  v_sem.at[slot]).wait()

    @pl.when(b + 1 < B)
    def _prefetch_next():
        issue(b + 1, 1 - slot)

    ncol = lax.broadcasted_iota(jnp.int32, (1, TN), 1)
    nbias = jnp.where(ncol < nc, 0.0, NEG)
    scale = 1.0 / 8.0
    rowid = lax.broadcasted_iota(jnp.int32, (TN, 1), 0)

    def body(W):
        if W > 0:
            pvalid = lax.broadcasted_iota(jnp.int32, (1, W), 1) < plen

        def scores_of(h):
            sl = slice(h * DH, (h + 1) * DH)
            qh = q[:, sl] * scale
            sn = lax.dot_general(qh, kn[:, sl], _DN,
                                 preferred_element_type=jnp.float32) + nbias
            if W == 0:
                return sn
            sp = jnp.dot(qh, ktb[slot, h, :, :W],
                         preferred_element_type=jnp.float32)
            sp = jnp.where(pvalid, sp, NEG)
            return jnp.concatenate([sp, sn], axis=1)

        def out_of(h, p, denom):
            sl = slice(h * DH, (h + 1) * DH)
            on = jnp.dot(p[:, W:], vn[:, sl],
                         preferred_element_type=jnp.float32)
            if W > 0:
                on = on + lax.dot_general(p[:, :W], vtb[slot, h, :, :W], _DN,
                                          preferred_element_type=jnp.float32)
            return on * (1.0 / denom)

        outs = []
        for h0 in range(0, H, 2):
            s_a = scores_of(h0)
            s_b = scores_of(h0 + 1)
            m_a = jnp.max(s_a, axis=-1, keepdims=True)
            m_b = jnp.max(s_b, axis=-1, keepdims=True)
            p_a = jnp.exp(s_a - m_a)
            p_b = jnp.exp(s_b - m_b)
            d_a = jnp.sum(p_a, axis=-1, keepdims=True)
            d_b = jnp.sum(p_b, axis=-1, keepdims=True)
            outs.append(out_of(h0, p_a, d_a))
            outs.append(out_of(h0 + 1, p_b, d_b))

        of = jnp.concatenate(outs, axis=-1)
        out = lax.dot_general(of, wo_ref[...], _DN,
                              preferred_element_type=jnp.float32) + bo_ref[...]
        o_ref[0] = jnp.where(rowid < nc, out, 0.0)

    for w in range(NC + 1):
        @pl.when(n_cur == w)
        def _(w=w):
            body(w * CP)


def kernel(x_new, rotary_cos, rotary_sin, past_k, past_v,
           Wq, bq, Wk, bk, Wv, bv, Wo, bo,
           past_lengths, new_token_counts, valid_new_mask):
    xf = x_new.reshape(B * TN, D)
    cosf = jnp.tile(rotary_cos.reshape(TN, DH), (B, 1))
    sin1 = rotary_sin.reshape(TN, DH)
    sinf = jnp.tile(jnp.concatenate([-sin1[:, :DH // 2], sin1[:, DH // 2:]],
                                    axis=-1), (B, 1))

    NJ = 4
    DJ = D // NJ
    q, kn, vn = pl.pallas_call(
        _proj_kernel,
        grid=(NJ,),
        in_specs=[
            pl.BlockSpec((B * TN, D), lambda j: (0, 0)),
            pl.BlockSpec((DJ, D), lambda j: (j, 0)),
            pl.BlockSpec((DJ, D), lambda j: (j, 0)),
            pl.BlockSpec((DJ, D), lambda j: (j, 0)),
            pl.BlockSpec((1, DJ), lambda j: (0, j)),
            pl.BlockSpec((1, DJ), lambda j: (0, j)),
            pl.BlockSpec((1, DJ), lambda j: (0, j)),
            pl.BlockSpec((B * TN, DH), lambda j: (0, 0)),
            pl.BlockSpec((B * TN, DH), lambda j: (0, 0)),
        ],
        out_specs=[pl.BlockSpec((B * TN, DJ), lambda j: (0, j))] * 3,
        out_shape=[jax.ShapeDtypeStruct((B * TN, D), jnp.float32)] * 3,
        compiler_params=pltpu.CompilerParams(
            dimension_semantics=("arbitrary",),
        ),
        name="qkv_rope_proj",
    )(xf, Wq, Wk, Wv, bq.reshape(1, D), bk.reshape(1, D), bv.reshape(1, D),
      cosf, sinf)

    lens = jnp.stack([past_lengths, new_token_counts]).astype(jnp.int32)

    out = pl.pallas_call(
        _attn_kernel,
        grid_spec=pltpu.PrefetchScalarGridSpec(
            num_scalar_prefetch=1,
            grid=(B,),
            in_specs=[
                pl.BlockSpec((1, TN, D), lambda b, lens: (b, 0, 0)),
                pl.BlockSpec((1, TN, D), lambda b, lens: (b, 0, 0)),
                pl.BlockSpec((1, TN, D), lambda b, lens: (b, 0, 0)),
                pl.BlockSpec(memory_space=pl.ANY),
                pl.BlockSpec(memory_space=pl.ANY),
                pl.BlockSpec((D, D), lambda b, lens: (0, 0)),
                pl.BlockSpec((1, D), lambda b, lens: (0, 0)),
            ],
            out_specs=pl.BlockSpec((1, TN, D), lambda b, lens: (b, 0, 0)),
            scratch_shapes=[
                pltpu.VMEM((2, H, DH, LP), jnp.float32),
                pltpu.VMEM((2, H, DH, LP), jnp.float32),
                pltpu.SemaphoreType.DMA((2,)),
                pltpu.SemaphoreType.DMA((2,)),
            ],
        ),
        out_shape=jax.ShapeDtypeStruct((B, TN, D), jnp.float32),
        compiler_params=pltpu.CompilerParams(
            dimension_semantics=("arbitrary",),
            vmem_limit_bytes=50 * 1024 * 1024,
        ),
        name="ragged_attn",
    )(lens, q.reshape(B, TN, D), kn.reshape(B, TN, D), vn.reshape(B, TN, D),
      jnp.swapaxes(past_k, 2, 3), jnp.swapaxes(past_v, 2, 3),
      Wo, bo.reshape(1, D))

    return out
